# Initial kernel scaffold; baseline (speedup 1.0000x reference)
import jax
import jax.numpy as jnp
from jax.experimental import pallas as pl


def kernel(seq, token_table, pos_table):
    raise NotImplementedError("write your pallas kernel here")


# speedup vs baseline: 3.6896x; 3.6896x over previous
import functools

import jax
import jax.numpy as jnp
from jax import lax
from jax.experimental import pallas as pl
from jax.experimental.pallas import tpu as pltpu
from jax.experimental.pallas import tpu_sc as plsc

NC = 2
NS = 16
NW = NC * NS

G = 100


def _make_kernel(B, L, V, D, chunk_seq):
  N = B * L
  rows_per_worker = N // NW
  chunk_rows = chunk_seq * L
  n_chunks = rows_per_worker // chunk_rows
  subg = chunk_rows // G
  g_per_worker = rows_per_worker // G
  nseg = D // 16

  mesh = plsc.VectorSubcoreMesh(
      core_axis_name="c", subcore_axis_name="s", num_cores=NC,
      num_subcores=NS)

  @functools.partial(
      pl.kernel,
      out_type=jax.ShapeDtypeStruct((N, D), jnp.float32),
      mesh=mesh,
      scratch_types=[
          pltpu.VMEM((subg, G), jnp.int32),
          pltpu.VMEM((chunk_rows, D), jnp.float32),
          pltpu.VMEM((L, D), jnp.float32),
          pltpu.SemaphoreType.DMA,
      ],
      compiler_params=pltpu.CompilerParams(use_tc_tiling_on_sc=False),
  )
  def k(idx_hbm, tok_hbm, pos_hbm, out_hbm, idx_v, rows_v, pos_v, gsem):
    cid = lax.axis_index("c")
    sid = lax.axis_index("s")
    wid = sid * NC + cid
    pltpu.sync_copy(pos_hbm, pos_v)
    base_row = wid * rows_per_worker
    base_g = wid * g_per_worker

    def chunk_body(c, carry):
      pltpu.sync_copy(idx_hbm.at[pl.ds(base_g + c * subg, subg)], idx_v)
      cps = [
          pltpu.async_copy(tok_hbm.at[idx_v.at[j]],
                           rows_v.at[pl.ds(j * G, G)], gsem)
          for j in range(subg)
      ]
      for cp in cps:
        cp.wait()

      def l_body(l, carry2):
        for seg in range(nseg):
          p = pos_v[l, pl.ds(seg * 16, 16)]
          for s in range(chunk_seq):
            plsc.addupdate(rows_v.at[s * L + l, pl.ds(seg * 16, 16)], p)
        return carry2

      lax.fori_loop(0, L, l_body, 0)
      pltpu.sync_copy(rows_v,
                      out_hbm.at[pl.ds(base_row + c * chunk_rows, chunk_rows)])
      return carry

    lax.fori_loop(0, n_chunks, chunk_body, 0)

  return k


def kernel(seq, token_table, pos_table):
  B, L = seq.shape
  V, D = token_table.shape
  N = B * L
  idx = seq.reshape(N // G, G).astype(jnp.int32)
  k = _make_kernel(B, L, V, D, chunk_seq=4)
  out = k(idx, token_table, pos_table)
  return out.reshape(B, L, D)


# speedup vs baseline: 4.1630x; 1.1283x over previous
import functools

import jax
import jax.numpy as jnp
from jax import lax
from jax.experimental import pallas as pl
from jax.experimental.pallas import tpu as pltpu
from jax.experimental.pallas import tpu_sc as plsc

NC = 2
NS = 16
NW = NC * NS

G = 100


def _make_kernel(B, L, V, D):
  chunk_seq = 2
  N = B * L
  rows_per_worker = N // NW
  chunk_rows = chunk_seq * L
  n_chunks = rows_per_worker // chunk_rows
  subg = chunk_rows // G
  idx_rows = rows_per_worker // G
  nseg = D // 16

  mesh = plsc.VectorSubcoreMesh(
      core_axis_name="c", subcore_axis_name="s", num_cores=NC,
      num_subcores=NS)

  @functools.partial(
      pl.kernel,
      out_type=jax.ShapeDtypeStruct((N, D), jnp.float32),
      mesh=mesh,
      scratch_types=[
          pltpu.VMEM((idx_rows, G), jnp.int32),
          pltpu.VMEM((2, chunk_rows, D), jnp.float32),
          pltpu.VMEM((L, D), jnp.float32),
          pltpu.SemaphoreType.DMA((2,)),
          pltpu.SemaphoreType.DMA((2,)),
      ],
      compiler_params=pltpu.CompilerParams(use_tc_tiling_on_sc=False),
  )
  def k(idx_hbm, tok_hbm, pos_hbm, out_hbm, idx_v, rows_v, pos_v, gsem, ssem):
    cid = lax.axis_index("c")
    sid = lax.axis_index("s")
    wid = sid * NC + cid
    base_row = wid * rows_per_worker
    pltpu.sync_copy(pos_hbm, pos_v)
    pltpu.sync_copy(idx_hbm.at[pl.ds(wid * idx_rows, idx_rows)], idx_v)

    def fire(c, buf):
      for j in range(subg):
        pltpu.async_copy(tok_hbm.at[idx_v.at[c * subg + j]],
                         rows_v.at[buf, pl.ds(j * G, G)], gsem.at[buf])

    def drain_gather(buf):
      pltpu.make_async_copy(tok_hbm.at[pl.ds(0, chunk_rows)],
                            rows_v.at[buf], gsem.at[buf]).wait()

    def drain_store(buf):
      pltpu.make_async_copy(rows_v.at[buf],
                            out_hbm.at[pl.ds(0, chunk_rows)],
                            ssem.at[buf]).wait()

    fire(0, 0)

    def body(c, carry):
      buf = lax.rem(c, 2)
      nbuf = 1 - buf

      @pl.when(c > 0)
      def _():
        drain_store(nbuf)

      @pl.when(c < n_chunks - 1)
      def _():
        fire(c + 1, nbuf)

      drain_gather(buf)

      @plsc.parallel_loop(0, L, 1, unroll=4)
      def _(l):
        for seg in range(nseg):
          p = pos_v[l, pl.ds(seg * 16, 16)]
          for s in range(chunk_seq):
            plsc.addupdate(rows_v.at[buf, s * L + l, pl.ds(seg * 16, 16)], p)

      pltpu.async_copy(
          rows_v.at[buf],
          out_hbm.at[pl.ds(base_row + c * chunk_rows, chunk_rows)],
          ssem.at[buf])
      return carry

    lax.fori_loop(0, n_chunks, body, 0)
    drain_store((n_chunks - 1) % 2)

  return k


def kernel(seq, token_table, pos_table):
  B, L = seq.shape
  V, D = token_table.shape
  N = B * L
  idx = seq.reshape(N // G, G).astype(jnp.int32)
  k = _make_kernel(B, L, V, D)
  out = k(idx, token_table, pos_table)
  return out.reshape(B, L, D)


# speedup vs baseline: 7.2417x; 1.7395x over previous
import functools

import jax
import jax.numpy as jnp
from jax import lax
from jax.experimental import pallas as pl
from jax.experimental.pallas import tpu as pltpu
from jax.experimental.pallas import tpu_sc as plsc

NC = 2
NS = 16
NW = NC * NS

G = 100


def _make_kernel(B, L, V, D):
  chunk_seq = 2
  N = B * L
  rows_per_worker = N // NW
  chunk_rows = chunk_seq * L
  n_chunks = rows_per_worker // chunk_rows
  subg = chunk_rows // G
  idx_rows = rows_per_worker // G
  nseg = D // 16

  mesh = plsc.VectorSubcoreMesh(
      core_axis_name="c", subcore_axis_name="s", num_cores=NC,
      num_subcores=NS)

  @functools.partial(
      pl.kernel,
      out_type=jax.ShapeDtypeStruct((N, 2 * D), jnp.float32),
      mesh=mesh,
      scratch_types=[
          pltpu.VMEM((idx_rows, G), jnp.int32),
          pltpu.VMEM((2, chunk_rows, D), jnp.float32),
          pltpu.VMEM((L, D), jnp.float32),
          pltpu.SemaphoreType.DMA((2,)),
          pltpu.SemaphoreType.DMA((2,)),
      ],
      compiler_params=pltpu.CompilerParams(use_tc_tiling_on_sc=False),
  )
  def k(idx_hbm, tok_hbm, pos_hbm, out_hbm, idx_v, rows_v, pos_v, gsem, ssem):
    cid = lax.axis_index("c")
    sid = lax.axis_index("s")
    wid = sid * NC + cid
    base_row = wid * rows_per_worker
    pltpu.sync_copy(pos_hbm, pos_v)
    pltpu.sync_copy(idx_hbm.at[pl.ds(wid * idx_rows, idx_rows)], idx_v)

    def fire(c, buf):
      for j in range(subg):
        pltpu.async_copy(tok_hbm.at[idx_v.at[c * subg + j]],
                         rows_v.at[buf, pl.ds(j * G, G)], gsem.at[buf])

    def drain_gather(buf):
      pltpu.make_async_copy(tok_hbm.at[pl.ds(0, chunk_rows)],
                            rows_v.at[buf], gsem.at[buf]).wait()

    def drain_store(buf):
      pltpu.make_async_copy(rows_v.at[buf],
                            out_hbm.at[pl.ds(0, chunk_rows), pl.ds(0, D)],
                            ssem.at[buf]).wait()

    fire(0, 0)

    def body(c, carry):
      buf = lax.rem(c, 2)
      nbuf = 1 - buf

      @pl.when(c > 0)
      def _():
        drain_store(nbuf)

      @pl.when(c < n_chunks - 1)
      def _():
        fire(c + 1, nbuf)

      drain_gather(buf)

      @plsc.parallel_loop(0, L, 1, unroll=4)
      def _(l):
        for seg in range(nseg):
          p = pos_v[l, pl.ds(seg * 16, 16)]
          for s in range(chunk_seq):
            plsc.addupdate(rows_v.at[buf, s * L + l, pl.ds(seg * 16, 16)], p)

      pltpu.async_copy(
          rows_v.at[buf],
          out_hbm.at[pl.ds(base_row + c * chunk_rows, chunk_rows), pl.ds(0, D)],
          ssem.at[buf])
      return carry

    lax.fori_loop(0, n_chunks, body, 0)
    drain_store((n_chunks - 1) % 2)

  return k


def kernel(seq, token_table, pos_table):
  B, L = seq.shape
  V, D = token_table.shape
  N = B * L
  idx = seq.reshape(N // G, G).astype(jnp.int32)
  k = _make_kernel(B, L, V, D)
  out = k(idx, token_table, pos_table)
  return out[:, :D].reshape(B, L, D)
